# Initial kernel scaffold; baseline (speedup 1.0000x reference)
import jax
import jax.numpy as jnp
from jax.experimental import pallas as pl


def kernel(flat_values, cu_seqlens):
    raise NotImplementedError("write your pallas kernel here")


# speedup vs baseline: 21.0032x; 21.0032x over previous
import functools

import jax
import jax.numpy as jnp
from jax import lax
from jax.experimental import pallas as pl
from jax.experimental.pallas import tpu as pltpu
from jax.experimental.pallas import tpu_sc as plsc

_FIXEDLEN = 4096
_PAD_VALUE = 1.0
_B = 16
_NW = 32
_CHUNK = (_B * _FIXEDLEN) // _NW
_LANES = 16
_BUF = _CHUNK + _LANES
_SRC_PAD = 32768 + _CHUNK + _BUF


def _make_densify():
    mesh = plsc.VectorSubcoreMesh(core_axis_name="c", subcore_axis_name="s")

    @functools.partial(
        pl.kernel,
        mesh=mesh,
        out_type=jax.ShapeDtypeStruct((_NW, _CHUNK), jnp.float32),
        scratch_types=[
            pltpu.VMEM((32,), jnp.int32),
            pltpu.VMEM((_BUF,), jnp.float32),
            pltpu.VMEM((_CHUNK,), jnp.float32),
        ],
    )
    def densify(flat_hbm, cu_hbm, out_hbm, cu_v, buf_v, out_v):
        wid = lax.axis_index("s") * 2 + lax.axis_index("c")
        row = wid // 2
        col0 = (wid % 2) * _CHUNK

        pltpu.sync_copy(cu_hbm, cu_v)
        cuw = cu_v[pl.ds(row, _LANES)]
        start = cuw[0]
        seglen = cuw[1] - start

        base = start + col0
        aligned = pl.multiple_of((base // 8) * 8, 8)
        r = base - aligned
        pltpu.sync_copy(flat_hbm.at[pl.ds(aligned, _BUF)], buf_v)

        lim = seglen - col0

        def body(j, carry):
            off = j * _LANES
            vals = buf_v[pl.ds(r + off, _LANES)]
            pos = lax.iota(jnp.int32, _LANES) + off
            out_v[pl.ds(off, _LANES)] = jnp.where(
                pos < lim, vals, jnp.float32(_PAD_VALUE))
            return carry

        lax.fori_loop(0, _CHUNK // _LANES, body, 0)
        pltpu.sync_copy(out_v, out_hbm.at[wid])

    return densify


_densify = _make_densify()


def kernel(flat_values, cu_seqlens):
    total = flat_values.shape[0]
    flat_padded = jnp.concatenate(
        [flat_values, jnp.zeros((_SRC_PAD - total,), jnp.float32)])
    cu_padded = jnp.concatenate(
        [cu_seqlens.astype(jnp.int32), jnp.zeros((32 - cu_seqlens.shape[0],), jnp.int32)])
    out = _densify(flat_padded, cu_padded)
    return out.reshape(_B, _FIXEDLEN)


# speedup vs baseline: 21.9817x; 1.0466x over previous
import functools

import jax
import jax.numpy as jnp
from jax import lax
from jax.experimental import pallas as pl
from jax.experimental.pallas import tpu as pltpu
from jax.experimental.pallas import tpu_sc as plsc

_FIXEDLEN = 4096
_PAD_VALUE = 1.0
_B = 16
_NW = 32
_CHUNK = (_B * _FIXEDLEN) // _NW
_LANES = 16
_WIN = _CHUNK + _LANES
_BUF = 2 * _CHUNK + _WIN + 2 * _LANES


@functools.lru_cache(maxsize=None)
def _make_densify(total: int):
    mesh = plsc.VectorSubcoreMesh(core_axis_name="c", subcore_axis_name="s")

    @functools.partial(
        pl.kernel,
        mesh=mesh,
        out_type=jax.ShapeDtypeStruct((_NW, _CHUNK), jnp.float32),
        scratch_types=[
            pltpu.VMEM((32,), jnp.int32),
            pltpu.VMEM((_BUF,), jnp.float32),
            pltpu.VMEM((_CHUNK,), jnp.float32),
        ],
    )
    def densify(flat_hbm, cu_hbm, out_hbm, cu_v, buf_v, out_v):
        wid = lax.axis_index("s") * 2 + lax.axis_index("c")
        row = wid // 2
        col0 = (wid % 2) * _CHUNK

        pltpu.sync_copy(cu_hbm, cu_v.at[pl.ds(0, _B + 1)])
        cuw = cu_v[pl.ds(row, _LANES)]
        start = cuw[0]
        seglen = cuw[1] - start

        base = start + col0
        aligned = jnp.minimum((base // 8) * 8, total - _WIN)
        win = pl.multiple_of(aligned, 8)
        r = base - win
        pltpu.sync_copy(flat_hbm.at[pl.ds(win, _WIN)], buf_v.at[pl.ds(0, _WIN)])

        lim = seglen - col0

        def body(j, carry):
            off = j * _LANES
            vals = buf_v[pl.ds(r + off, _LANES)]
            pos = lax.iota(jnp.int32, _LANES) + off
            out_v[pl.ds(off, _LANES)] = jnp.where(
                pos < lim, vals, jnp.float32(_PAD_VALUE))
            return carry

        lax.fori_loop(0, _CHUNK // _LANES, body, 0, unroll=8)
        pltpu.sync_copy(out_v, out_hbm.at[wid])

    return densify


def kernel(flat_values, cu_seqlens):
    out = _make_densify(flat_values.shape[0])(
        flat_values, cu_seqlens.astype(jnp.int32))
    return out.reshape(_B, _FIXEDLEN)


# speedup vs baseline: 23.1054x; 1.0511x over previous
import functools

import jax
import jax.numpy as jnp
from jax import lax
from jax.experimental import pallas as pl
from jax.experimental.pallas import tpu as pltpu
from jax.experimental.pallas import tpu_sc as plsc

_FIXEDLEN = 4096
_PAD_VALUE = 1.0
_B = 16
_NW = 32
_CHUNK = (_B * _FIXEDLEN) // _NW
_HALF = _CHUNK // 2
_LANES = 16
_WIN = _CHUNK + _LANES
_BUF = 2 * _CHUNK + _WIN + 2 * _LANES


@functools.lru_cache(maxsize=None)
def _make_densify(total: int):
    mesh = plsc.VectorSubcoreMesh(core_axis_name="c", subcore_axis_name="s")

    @functools.partial(
        pl.kernel,
        mesh=mesh,
        out_type=jax.ShapeDtypeStruct((_B, _FIXEDLEN), jnp.float32),
        scratch_types=[
            pltpu.VMEM((32,), jnp.int32),
            pltpu.VMEM((_BUF,), jnp.float32),
            pltpu.VMEM((_CHUNK,), jnp.float32),
            pltpu.SemaphoreType.DMA,
        ],
    )
    def densify(flat_hbm, cu_hbm, out_hbm, cu_v, buf_v, out_v, sem):
        wid = lax.axis_index("s") * 2 + lax.axis_index("c")
        row = wid // 2
        col0 = pl.multiple_of((wid % 2) * _CHUNK, _CHUNK)

        pltpu.sync_copy(cu_hbm, cu_v.at[pl.ds(0, _B + 1)])
        cuw = cu_v[pl.ds(row, _LANES)]
        start = cuw[0]
        seglen = cuw[1] - start

        base = start + col0
        aligned = jnp.minimum((base // 8) * 8, total - _WIN)
        win = pl.multiple_of(aligned, 8)
        r = base - win
        pltpu.sync_copy(flat_hbm.at[pl.ds(win, _WIN)], buf_v.at[pl.ds(0, _WIN)])

        lim = seglen - col0

        def body(j, carry):
            off = j * _LANES
            vals = buf_v[pl.ds(r + off, _LANES)]
            pos = lax.iota(jnp.int32, _LANES) + off
            out_v[pl.ds(off, _LANES)] = jnp.where(
                pos < lim, vals, jnp.float32(_PAD_VALUE))
            return carry

        lax.fori_loop(0, _HALF // _LANES, body, 0, unroll=16)
        h0 = pltpu.make_async_copy(
            out_v.at[pl.ds(0, _HALF)],
            out_hbm.at[row, pl.ds(col0, _HALF)], sem)
        h0.start()
        lax.fori_loop(_HALF // _LANES, _CHUNK // _LANES, body, 0, unroll=16)
        h1 = pltpu.make_async_copy(
            out_v.at[pl.ds(_HALF, _HALF)],
            out_hbm.at[row, pl.ds(col0 + _HALF, _HALF)], sem)
        h1.start()
        h0.wait()
        h1.wait()

    return densify


def kernel(flat_values, cu_seqlens):
    return _make_densify(flat_values.shape[0])(
        flat_values, cu_seqlens.astype(jnp.int32))


# speedup vs baseline: 23.8017x; 1.0301x over previous
import functools

import jax
import jax.numpy as jnp
from jax import lax
from jax.experimental import pallas as pl
from jax.experimental.pallas import tpu as pltpu
from jax.experimental.pallas import tpu_sc as plsc

_FIXEDLEN = 4096
_PAD_VALUE = 1.0
_B = 16
_NW = 32
_CHUNK = (_B * _FIXEDLEN) // _NW
_HALF = _CHUNK // 2
_LANES = 16
_WIN = _CHUNK + _LANES
_BUF = 2 * _CHUNK + _WIN + 2 * _LANES


@functools.lru_cache(maxsize=None)
def _make_densify(total: int):
    mesh = plsc.VectorSubcoreMesh(core_axis_name="c", subcore_axis_name="s")

    @functools.partial(
        pl.kernel,
        mesh=mesh,
        out_type=jax.ShapeDtypeStruct((_B, _FIXEDLEN), jnp.float32),
        scratch_types=[
            pltpu.VMEM((32,), jnp.int32),
            pltpu.VMEM((_BUF,), jnp.float32),
            pltpu.VMEM((_CHUNK,), jnp.float32),
        ],
    )
    def densify(flat_hbm, cu_hbm, out_hbm, cu_v, buf_v, out_v):
        wid = lax.axis_index("s") * 2 + lax.axis_index("c")
        row = wid // 2
        col0 = pl.multiple_of((wid % 2) * _CHUNK, _CHUNK)

        pltpu.sync_copy(cu_hbm, cu_v.at[pl.ds(0, _B + 1)])
        cuw = cu_v[pl.ds(row, _LANES)]
        start = cuw[0]
        seglen = cuw[1] - start

        base = start + col0
        aligned = jnp.minimum((base // 8) * 8, total - _WIN)
        win = pl.multiple_of(aligned, 8)
        r = base - win
        pltpu.sync_copy(flat_hbm.at[pl.ds(win, _WIN)], buf_v.at[pl.ds(0, _WIN)])

        lim = seglen - col0

        def body(j, carry):
            off = j * _LANES
            vals = buf_v[pl.ds(r + off, _LANES)]
            pos = lax.iota(jnp.int32, _LANES) + off
            out_v[pl.ds(off, _LANES)] = jnp.where(
                pos < lim, vals, jnp.float32(_PAD_VALUE))
            return carry

        lax.fori_loop(0, _CHUNK // _LANES, body, 0, unroll=4)
        pltpu.sync_copy(out_v, out_hbm.at[row, pl.ds(col0, _CHUNK)])

    return densify


def kernel(flat_values, cu_seqlens):
    return _make_densify(flat_values.shape[0])(
        flat_values, cu_seqlens.astype(jnp.int32))


# speedup vs baseline: 23.9216x; 1.0050x over previous
import functools

import jax
import jax.numpy as jnp
from jax import lax
from jax.experimental import pallas as pl
from jax.experimental.pallas import tpu as pltpu
from jax.experimental.pallas import tpu_sc as plsc

_FIXEDLEN = 4096
_PAD_VALUE = 1.0
_B = 16
_NW = 32
_CHUNK = (_B * _FIXEDLEN) // _NW
_HALF = _CHUNK // 2
_LANES = 16
_WIN = _CHUNK + _LANES
_BUF = 2 * _CHUNK + _WIN + 2 * _LANES


@functools.lru_cache(maxsize=None)
def _make_densify(total: int):
    mesh = plsc.VectorSubcoreMesh(core_axis_name="c", subcore_axis_name="s")

    @functools.partial(
        pl.kernel,
        mesh=mesh,
        out_type=jax.ShapeDtypeStruct((_B, _FIXEDLEN), jnp.float32),
        scratch_types=[
            pltpu.VMEM((32,), jnp.int32),
            pltpu.VMEM((_BUF,), jnp.float32),
            pltpu.VMEM((_CHUNK,), jnp.float32),
        ],
    )
    def densify(flat_hbm, cu_hbm, out_hbm, cu_v, buf_v, out_v):
        wid = lax.axis_index("s") * 2 + lax.axis_index("c")
        row = wid // 2
        col0 = pl.multiple_of((wid % 2) * _CHUNK, _CHUNK)

        pltpu.sync_copy(cu_hbm, cu_v.at[pl.ds(0, _B + 1)])
        cuw = cu_v[pl.ds(row, _LANES)]
        start = cuw[0]
        seglen = cuw[1] - start

        base = start + col0
        aligned = jnp.minimum((base // 8) * 8, total - _WIN)
        win = pl.multiple_of(aligned, 8)
        r = base - win
        pltpu.sync_copy(flat_hbm.at[pl.ds(win, _WIN)], buf_v.at[pl.ds(0, _WIN)])

        lim = seglen - col0

        def body(j, carry):
            off = j * _LANES
            vals = buf_v[pl.ds(r + off, _LANES)]
            pos = lax.iota(jnp.int32, _LANES) + off
            out_v[pl.ds(off, _LANES)] = jnp.where(
                pos < lim, vals, jnp.float32(_PAD_VALUE))
            return carry

        lax.fori_loop(0, _CHUNK // _LANES, body, 0, unroll=2)
        pltpu.sync_copy(out_v, out_hbm.at[row, pl.ds(col0, _CHUNK)])

    return densify


def kernel(flat_values, cu_seqlens):
    return _make_densify(flat_values.shape[0])(
        flat_values, cu_seqlens.astype(jnp.int32))


# speedup vs baseline: 24.2548x; 1.0139x over previous
import functools

import jax
import jax.numpy as jnp
from jax import lax
from jax.experimental import pallas as pl
from jax.experimental.pallas import tpu as pltpu
from jax.experimental.pallas import tpu_sc as plsc

_FIXEDLEN = 4096
_PAD_VALUE = 1.0
_B = 16
_NW = 32
_CHUNK = (_B * _FIXEDLEN) // _NW
_HALF = _CHUNK // 2
_LANES = 16
_WIN = _CHUNK + _LANES
_BUF = 2 * _CHUNK + _WIN + 2 * _LANES


@functools.lru_cache(maxsize=None)
def _make_densify(total: int):
    mesh = plsc.VectorSubcoreMesh(core_axis_name="c", subcore_axis_name="s")

    @functools.partial(
        pl.kernel,
        mesh=mesh,
        out_type=jax.ShapeDtypeStruct((_B, _FIXEDLEN), jnp.float32),
        scratch_types=[
            pltpu.VMEM((32,), jnp.int32),
            pltpu.VMEM((_BUF,), jnp.float32),
            pltpu.VMEM((_CHUNK,), jnp.float32),
        ],
    )
    def densify(flat_hbm, cu_hbm, out_hbm, cu_v, buf_v, out_v):
        wid = lax.axis_index("s") * 2 + lax.axis_index("c")
        row = wid // 2
        col0 = pl.multiple_of((wid % 2) * _CHUNK, _CHUNK)

        pltpu.sync_copy(cu_hbm, cu_v.at[pl.ds(0, _B + 1)])
        cuw = cu_v[pl.ds(row, _LANES)]
        start = cuw[0]
        seglen = cuw[1] - start

        base = start + col0
        aligned = jnp.minimum((base // 8) * 8, total - _WIN)
        win = pl.multiple_of(aligned, 8)
        r = base - win
        pltpu.sync_copy(flat_hbm.at[pl.ds(win, _WIN)], buf_v.at[pl.ds(0, _WIN)])

        lim = seglen - col0

        @plsc.parallel_loop(0, _CHUNK, _LANES, unroll=4)
        def body(off):
            vals = buf_v[pl.ds(r + off, _LANES)]
            pos = lax.iota(jnp.int32, _LANES) + off
            out_v[pl.ds(off, _LANES)] = jnp.where(
                pos < lim, vals, jnp.float32(_PAD_VALUE))
        pltpu.sync_copy(out_v, out_hbm.at[row, pl.ds(col0, _CHUNK)])

    return densify


def kernel(flat_values, cu_seqlens):
    return _make_densify(flat_values.shape[0])(
        flat_values, cu_seqlens.astype(jnp.int32))


# speedup vs baseline: 24.3710x; 1.0048x over previous
import functools

import jax
import jax.numpy as jnp
from jax import lax
from jax.experimental import pallas as pl
from jax.experimental.pallas import tpu as pltpu
from jax.experimental.pallas import tpu_sc as plsc

_FIXEDLEN = 4096
_PAD_VALUE = 1.0
_B = 16
_NW = 32
_CHUNK = (_B * _FIXEDLEN) // _NW
_HALF = _CHUNK // 2
_LANES = 16
_WIN = _CHUNK + _LANES
_BUF = 2 * _CHUNK + _WIN + 2 * _LANES


@functools.lru_cache(maxsize=None)
def _make_densify(total: int):
    mesh = plsc.VectorSubcoreMesh(core_axis_name="c", subcore_axis_name="s")

    @functools.partial(
        pl.kernel,
        mesh=mesh,
        out_type=jax.ShapeDtypeStruct((_B, _FIXEDLEN), jnp.float32),
        scratch_types=[
            pltpu.VMEM((32,), jnp.int32),
            pltpu.VMEM((_BUF,), jnp.float32),
            pltpu.VMEM((_CHUNK,), jnp.float32),
            pltpu.SemaphoreType.DMA,
        ],
    )
    def densify(flat_hbm, cu_hbm, out_hbm, cu_v, buf_v, out_v, sem):
        wid = lax.axis_index("s") * 2 + lax.axis_index("c")
        row = wid // 2
        col0 = pl.multiple_of((wid % 2) * _CHUNK, _CHUNK)

        pltpu.sync_copy(cu_hbm, cu_v.at[pl.ds(0, _B + 1)])
        cuw = cu_v[pl.ds(row, _LANES)]
        start = cuw[0]
        seglen = cuw[1] - start

        base = start + col0
        aligned = jnp.minimum((base // 8) * 8, total - _WIN)
        win = pl.multiple_of(aligned, 8)
        r = base - win
        pltpu.sync_copy(flat_hbm.at[pl.ds(win, _WIN)], buf_v.at[pl.ds(0, _WIN)])

        lim = seglen - col0

        def half(lo):
            @plsc.parallel_loop(lo, lo + _HALF, _LANES, unroll=4)
            def body(off):
                vals = buf_v[pl.ds(r + off, _LANES)]
                pos = lax.iota(jnp.int32, _LANES) + off
                out_v[pl.ds(off, _LANES)] = jnp.where(
                    pos < lim, vals, jnp.float32(_PAD_VALUE))

        half(0)
        h0 = pltpu.make_async_copy(
            out_v.at[pl.ds(0, _HALF)],
            out_hbm.at[row, pl.ds(col0, _HALF)], sem)
        h0.start()
        half(_HALF)
        h1 = pltpu.make_async_copy(
            out_v.at[pl.ds(_HALF, _HALF)],
            out_hbm.at[row, pl.ds(col0 + _HALF, _HALF)], sem)
        h1.start()
        h0.wait()
        h1.wait()

    return densify


def kernel(flat_values, cu_seqlens):
    return _make_densify(flat_values.shape[0])(
        flat_values, cu_seqlens.astype(jnp.int32))
